# Initial kernel scaffold; baseline (speedup 1.0000x reference)
import jax
import jax.numpy as jnp
from jax.experimental import pallas as pl


def kernel(X, W, rows, cols, vals):
    raise NotImplementedError("write your pallas kernel here")


# speedup vs baseline: 5.4007x; 5.4007x over previous
import functools

import jax
import jax.numpy as jnp
from jax import lax
from jax.experimental import pallas as pl
from jax.experimental.pallas import tpu as pltpu
from jax.experimental.pallas import tpu_sc as plsc

_NC = 2
_NS = 16
_LANES = 16
_CHUNK = 128


def _spmm_partials(X, rows, cols, vals):
    N, D = X.shape
    E = rows.shape[0]
    assert E % _CHUNK == 0
    n_chunks = E // _CHUNK
    rows_per_sub = (N // _NS) // 8 * 8
    tail_base = _NS * rows_per_sub
    tail_rows = N - tail_base

    mesh = plsc.VectorSubcoreMesh(core_axis_name="c", subcore_axis_name="s")

    @functools.partial(
        pl.kernel,
        out_type=jax.ShapeDtypeStruct((_NC, N, D), jnp.float32),
        mesh=mesh,
        scratch_types=[
            pltpu.VMEM((_CHUNK,), jnp.int32),
            pltpu.VMEM((_CHUNK,), jnp.int32),
            pltpu.VMEM((_CHUNK,), jnp.float32),
            pltpu.VMEM((_CHUNK, D), jnp.float32),
            pltpu.VMEM_SHARED((N, D), jnp.float32),
            pltpu.SemaphoreType.DMA,
        ],
    )
    def sc_kernel(x_hbm, zeros_hbm, rows_hbm, cols_hbm, vals_hbm, out_hbm,
                  cols_v, rows_v, vals_v, buf_v, acc_sh, sem):
        cc = lax.axis_index("c")
        ss = lax.axis_index("s")
        base = ss * rows_per_sub

        pltpu.sync_copy(zeros_hbm.at[pl.ds(base, rows_per_sub)],
                        acc_sh.at[pl.ds(base, rows_per_sub)])
        if tail_rows:
            @pl.when(ss == _NS - 1)
            def _():
                pltpu.sync_copy(zeros_hbm.at[pl.ds(tail_base, tail_rows)],
                                acc_sh.at[pl.ds(tail_base, tail_rows)])
        plsc.subcore_barrier()

        wid = ss * _NC + cc

        @pl.loop(wid, n_chunks, step=_NC * _NS)
        def _(chunk):
            ebase = chunk * _CHUNK
            pltpu.sync_copy(cols_hbm.at[pl.ds(ebase, _CHUNK)], cols_v)
            pltpu.sync_copy(rows_hbm.at[pl.ds(ebase, _CHUNK)], rows_v)
            pltpu.sync_copy(vals_hbm.at[pl.ds(ebase, _CHUNK)], vals_v)
            pltpu.async_copy(x_hbm.at[cols_v], buf_v, sem).wait()

            @pl.loop(0, _CHUNK, step=_LANES)
            def _(e0):
                vv = vals_v[pl.ds(e0, _LANES)]
                for t in range(_LANES):
                    v = vv[t]
                    e = e0 + t
                    for j in range(0, D, _LANES):
                        buf_v[e, pl.ds(j, _LANES)] = (
                            buf_v[e, pl.ds(j, _LANES)] * v)

            pltpu.sync_copy(buf_v, acc_sh.at[rows_v], add=True)

        plsc.subcore_barrier()
        pltpu.sync_copy(acc_sh.at[pl.ds(base, rows_per_sub)],
                        out_hbm.at[cc, pl.ds(base, rows_per_sub)])
        if tail_rows:
            @pl.when(ss == _NS - 1)
            def _():
                pltpu.sync_copy(acc_sh.at[pl.ds(tail_base, tail_rows)],
                                out_hbm.at[cc, pl.ds(tail_base, tail_rows)])

    zeros = jnp.zeros((N, D), jnp.float32)
    return sc_kernel(X, zeros, rows, cols, vals)


def _finish(p0, p1, wt):
    N, D = p0.shape
    blk = 1000
    assert N % blk == 0

    def body(p0_ref, p1_ref, wt_ref, o_ref):
        acc = p0_ref[...] + p1_ref[...]
        o_ref[...] = jnp.dot(acc, wt_ref[...],
                             preferred_element_type=jnp.float32)

    return pl.pallas_call(
        body,
        grid=(N // blk,),
        in_specs=[
            pl.BlockSpec((blk, D), lambda i: (i, 0)),
            pl.BlockSpec((blk, D), lambda i: (i, 0)),
            pl.BlockSpec((D, D), lambda i: (0, 0)),
        ],
        out_specs=pl.BlockSpec((blk, D), lambda i: (i, 0)),
        out_shape=jax.ShapeDtypeStruct((N, D), jnp.float32),
    )(p0, p1, wt)


def kernel(X, W, rows, cols, vals):
    parts = _spmm_partials(
        X, rows.astype(jnp.int32), cols.astype(jnp.int32), vals)
    return _finish(parts[0], parts[1], W.T)


# speedup vs baseline: 9.9542x; 1.8431x over previous
import dataclasses
import functools

import jax
import jax.numpy as jnp
from jax import lax
from jax.experimental import pallas as pl
from jax.experimental.pallas import tpu as pltpu
from jax.experimental.pallas import tpu_sc as plsc

_NC = 2
_NS = 16
_NW = _NC * _NS
_LANES = 16
_CHUNK = 128


def _scale_rows(blk, buf):
    @pl.loop(0, _CHUNK, step=_LANES)
    def _(e0):
        vv = plsc.bitcast(blk[2, pl.ds(e0, _LANES)], jnp.float32)
        for t in range(_LANES):
            v = vv[t]
            e = e0 + t
            for j in range(0, 128, _LANES):
                buf[e, pl.ds(j, _LANES)] = buf[e, pl.ds(j, _LANES)] * v


def _spmm_partials(X, idx_blocks, n_chunks):
    N, D = X.shape
    assert D == 128
    per_tile = n_chunks // _NW
    n_main = per_tile * _NW
    n_left = n_chunks - n_main
    assert per_tile % 2 == 0 and n_left < _NW
    n_pairs = per_tile // 2

    rows_per_sub = (N // _NS) // 8 * 8
    tail_base = _NS * rows_per_sub
    tail_rows = N - tail_base

    mesh = plsc.VectorSubcoreMesh(core_axis_name="c", subcore_axis_name="s")
    cp = pltpu.CompilerParams()
    if "needs_layout_passes" in pltpu.CompilerParams.__dataclass_fields__:
        cp = dataclasses.replace(cp, needs_layout_passes=False)

    @functools.partial(
        pl.kernel,
        out_type=jax.ShapeDtypeStruct((_NC, N, D), jnp.float32),
        mesh=mesh,
        compiler_params=cp,
        scratch_types=[
            pltpu.VMEM((3, _CHUNK), jnp.int32),
            pltpu.VMEM((3, _CHUNK), jnp.int32),
            pltpu.VMEM((_CHUNK, 128), jnp.float32),
            pltpu.VMEM((_CHUNK, 128), jnp.float32),
            pltpu.VMEM_SHARED((N, 128), jnp.float32),
            pltpu.SemaphoreType.DMA,
            pltpu.SemaphoreType.DMA,
            pltpu.SemaphoreType.DMA,
            pltpu.SemaphoreType.DMA,
            pltpu.SemaphoreType.DMA,
            pltpu.SemaphoreType.DMA,
        ],
    )
    def sc_kernel(x_hbm, idx_hbm, out_hbm,
                  blk0, blk1, buf0, buf1, acc_sh,
                  sem_i0, sem_i1, sem_g0, sem_g1, sem_s0, sem_s1):
        cc = lax.axis_index("c")
        ss = lax.axis_index("s")
        wid = ss * _NC + cc
        base = ss * rows_per_sub

        zeros16 = jnp.zeros((_LANES,), jnp.float32)

        @pl.loop(0, _CHUNK)
        def _(r):
            for j in range(0, 128, _LANES):
                buf0[r, pl.ds(j, _LANES)] = zeros16

        off = 0
        while off < rows_per_sub:
            sz = min(_CHUNK, rows_per_sub - off)
            pltpu.sync_copy(buf0.at[pl.ds(0, sz)],
                            acc_sh.at[pl.ds(base + off, sz)])
            off += sz
        if tail_rows:
            @pl.when(ss == _NS - 1)
            def _():
                pltpu.sync_copy(buf0.at[pl.ds(0, tail_rows)],
                                acc_sh.at[pl.ds(tail_base, tail_rows)])
        plsc.subcore_barrier()

        def idx_start(k, blk, sem):
            return pltpu.async_copy(idx_hbm.at[wid + k * _NW], blk, sem)

        def idx_wait(blk, sem):
            pltpu.make_async_copy(idx_hbm.at[0], blk, sem).wait()

        def gather_start(blk, buf, sem):
            return pltpu.async_copy(x_hbm.at[blk.at[0]], buf, sem)

        def gather_wait(blk, buf, sem):
            pltpu.make_async_copy(x_hbm.at[blk.at[0]], buf, sem).wait()

        def scatter_start(blk, buf, sem):
            return pltpu.async_copy(buf, acc_sh.at[blk.at[1]], sem, add=True)

        idx_start(0, blk0, sem_i0).wait()
        gather_start(blk0, buf0, sem_g0)
        idx_start(1, blk1, sem_i1)

        @pl.loop(0, n_pairs)
        def _(it):
            not_last = it < n_pairs - 1
            idx_wait(blk1, sem_i1)
            h_g1 = gather_start(blk1, buf1, sem_g1)
            gather_wait(blk0, buf0, sem_g0)
            _scale_rows(blk0, buf0)
            h_s0 = scatter_start(blk0, buf0, sem_s0)
            h_g1.wait()
            _scale_rows(blk1, buf1)
            h_s0.wait()

            @pl.when(not_last)
            def _():
                idx_start(2 * it + 2, blk0, sem_i0)

            h_s1 = scatter_start(blk1, buf1, sem_s1)

            @pl.when(not_last)
            def _():
                idx_wait(blk0, sem_i0)
                gather_start(blk0, buf0, sem_g0)

            h_s1.wait()

            @pl.when(not_last)
            def _():
                idx_start(2 * it + 3, blk1, sem_i1)

        if n_left:
            @pl.when(wid < n_left)
            def _():
                pltpu.async_copy(idx_hbm.at[n_main + wid], blk0,
                                 sem_i0).wait()
                pltpu.async_copy(x_hbm.at[blk0.at[0]], buf0, sem_g0).wait()
                _scale_rows(blk0, buf0)
                pltpu.async_copy(buf0, acc_sh.at[blk0.at[1]], sem_s0,
                                 add=True).wait()

        plsc.subcore_barrier()

        pltpu.sync_copy(acc_sh.at[pl.ds(base, rows_per_sub)],
                        out_hbm.at[cc, pl.ds(base, rows_per_sub)])
        if tail_rows:
            @pl.when(ss == _NS - 1)
            def _():
                pltpu.sync_copy(acc_sh.at[pl.ds(tail_base, tail_rows)],
                                out_hbm.at[cc, pl.ds(tail_base, tail_rows)])

    return sc_kernel(X, idx_blocks)


def _finish(p0, p1, wt):
    N, D = p0.shape
    blk = 1000
    assert N % blk == 0

    def body(p0_ref, p1_ref, wt_ref, o_ref):
        acc = p0_ref[...] + p1_ref[...]
        o_ref[...] = jnp.dot(acc, wt_ref[...],
                             preferred_element_type=jnp.float32)

    return pl.pallas_call(
        body,
        grid=(N // blk,),
        in_specs=[
            pl.BlockSpec((blk, D), lambda i: (i, 0)),
            pl.BlockSpec((blk, D), lambda i: (i, 0)),
            pl.BlockSpec((D, D), lambda i: (0, 0)),
        ],
        out_specs=pl.BlockSpec((blk, D), lambda i: (i, 0)),
        out_shape=jax.ShapeDtypeStruct((N, D), jnp.float32),
    )(p0, p1, wt)


def kernel(X, W, rows, cols, vals):
    E = rows.shape[0]
    assert E % _CHUNK == 0
    n_chunks = E // _CHUNK
    idx_blocks = jnp.stack(
        [
            cols.astype(jnp.int32).reshape(n_chunks, _CHUNK),
            rows.astype(jnp.int32).reshape(n_chunks, _CHUNK),
            jax.lax.bitcast_convert_type(vals, jnp.int32).reshape(
                n_chunks, _CHUNK),
        ],
        axis=1,
    )
    parts = _spmm_partials(X, idx_blocks, n_chunks)
    return _finish(parts[0], parts[1], W.T)
